# baseline (speedup 1.0000x reference)
import functools

import jax
import jax.numpy as jnp
from jax import lax
from jax.experimental import pallas as pl
from jax.experimental.pallas import tpu as pltpu
from jax.experimental.pallas import tpu_sc as plsc

DIM = 64
BATCH = 16384
NUM_CORES = 2
NUM_SUBCORES = 16
NW = NUM_CORES * NUM_SUBCORES
B_PER_W = BATCH // NW
IDX_CHUNK = 128
N_CHUNKS = B_PER_W // IDX_CHUNK
GROUP = 16
N_GROUPS = B_PER_W // GROUP


def _mf_sc_kernel(model_hbm, prompt_hbm, p_tab, q_tab, w_hbm, b0_hbm, b1_hbm,
                  out0_hbm, out1_hbm,
                  midx_v, pidx_v, p_rows, q_rows, w_v, b0_v, b1_v,
                  out0_v, out1_v, sem):
    wid = lax.axis_index("s") * NUM_CORES + lax.axis_index("c")
    base = wid * B_PER_W

    pltpu.sync_copy(w_hbm, w_v)
    pltpu.sync_copy(b0_hbm, b0_v)
    pltpu.sync_copy(b1_hbm, b1_v)
    for j in range(N_CHUNKS):
        off = base + j * IDX_CHUNK
        pltpu.sync_copy(model_hbm.at[pl.ds(off, IDX_CHUNK)], midx_v.at[j])
        pltpu.sync_copy(prompt_hbm.at[pl.ds(off, IDX_CHUNK)], pidx_v.at[j])

    copies = []
    for j in range(N_CHUNKS):
        dst = pl.ds(j * IDX_CHUNK, IDX_CHUNK)
        copies.append(pltpu.async_copy(q_tab.at[pidx_v.at[j]],
                                       q_rows.at[dst], sem))
        copies.append(pltpu.async_copy(p_tab.at[midx_v.at[j]],
                                       p_rows.at[dst], sem))
    for cp in copies:
        cp.wait()

    w0 = [w_v[0, pl.ds(k * 16, 16)] for k in range(4)]
    w1 = [w_v[1, pl.ds(k * 16, 16)] for k in range(4)]
    b0 = b0_v[pl.ds(0, 16)]
    b1 = b1_v[pl.ds(0, 16)]
    lane = lax.iota(jnp.int32, 16)
    last = lane == 15

    def group_body(g, carry):
        for e in range(GROUP):
            i = g * GROUP + e
            h = [p_rows[i, pl.ds(k * 16, 16)] * q_rows[i, pl.ds(k * 16, 16)]
                 for k in range(4)]
            s0 = h[0] * w0[0] + h[1] * w0[1] + h[2] * w0[2] + h[3] * w0[3]
            s1 = h[0] * w1[0] + h[1] * w1[1] + h[2] * w1[2] + h[3] * w1[3]
            r0 = plsc.cumsum(s0) + b0
            r1 = plsc.cumsum(s1) + b1
            idx = jnp.full((16,), i, jnp.int32)
            plsc.store_scatter(out0_v, [idx], r0, mask=last)
            plsc.store_scatter(out1_v, [idx], r1, mask=last)
        return carry

    lax.fori_loop(0, N_GROUPS, group_body, 0)

    pltpu.sync_copy(out0_v, out0_hbm.at[pl.ds(base, B_PER_W)])
    pltpu.sync_copy(out1_v, out1_hbm.at[pl.ds(base, B_PER_W)])


@jax.jit
def kernel(model, prompt, P, Q, W, b):
    b0_splat = jnp.full((16,), b[0], jnp.float32)
    b1_splat = jnp.full((16,), b[1], jnp.float32)
    mesh = plsc.VectorSubcoreMesh(core_axis_name="c", subcore_axis_name="s")
    f = functools.partial(
        pl.kernel,
        out_type=[jax.ShapeDtypeStruct((BATCH,), jnp.float32),
                  jax.ShapeDtypeStruct((BATCH,), jnp.float32)],
        mesh=mesh,
        scratch_types=[
            pltpu.VMEM((N_CHUNKS, IDX_CHUNK), jnp.int32),
            pltpu.VMEM((N_CHUNKS, IDX_CHUNK), jnp.int32),
            pltpu.VMEM((B_PER_W, DIM), jnp.float32),
            pltpu.VMEM((B_PER_W, DIM), jnp.float32),
            pltpu.VMEM((2, DIM), jnp.float32),
            pltpu.VMEM((16,), jnp.float32),
            pltpu.VMEM((16,), jnp.float32),
            pltpu.VMEM((B_PER_W,), jnp.float32),
            pltpu.VMEM((B_PER_W,), jnp.float32),
            pltpu.SemaphoreType.DMA,
        ],
        compiler_params=pltpu.CompilerParams(needs_layout_passes=False,
                                             use_tc_tiling_on_sc=False),
    )(_mf_sc_kernel)
    out0, out1 = f(model, prompt, P, Q, W, b0_splat, b1_splat)
    return jnp.stack([out0, out1], axis=1)


# speedup vs baseline: 2.1221x; 2.1221x over previous
import functools

import jax
import jax.numpy as jnp
from jax import lax
from jax.experimental import pallas as pl
from jax.experimental.pallas import tpu as pltpu
from jax.experimental.pallas import tpu_sc as plsc

DIM = 64
BATCH = 16384
NUM_CORES = 2
NUM_SUBCORES = 16
NW = NUM_CORES * NUM_SUBCORES
B_PER_W = BATCH // NW
IDX_CHUNK = 128
N_CHUNKS = B_PER_W // IDX_CHUNK
GROUP = 16
N_GROUPS = B_PER_W // GROUP
NUM_PROMPTS_C = 1000000


N_TC = 7813
TC_PER_TILE = 245
CHUNK_TC = 4
CHUNK_COLS = CHUNK_TC * 128
N_STEPS = 62
N_PAIRS = N_STEPS // 2
S_CLAMP = N_TC - CHUNK_TC
HIT_CAP = 1024 + 32
CHIT_CAP = 96
QG_ROWS = BATCH + 16


def _q_extract_kernel(qt_hbm, prompt_hbm, qg_hbm,
                      pidx_v, hitc_v, hiti_v, chc_v, chi_v, blk_v, rowbuf_v,
                      semc0, semc1, semw):
    wid = lax.axis_index("s") * NUM_CORES + lax.axis_index("c")
    lane = lax.iota(jnp.int32, 16)
    lo_tc = wid * TC_PER_TILE
    lo = lo_tc * 128
    hi = jnp.minimum(lo + TC_PER_TILE * 128, NUM_PROMPTS_C)

    pltpu.sync_copy(prompt_hbm, pidx_v)

    def scan_body(v, cnt):
        c = pidx_v[pl.ds(v * 16, 16)]
        m = (c >= lo) & (c < hi)
        mi = m.astype(jnp.int32)
        pos = cnt + plsc.cumsum(mi) - mi
        plsc.store_scatter(hitc_v, [pos], c, mask=m)
        plsc.store_scatter(hiti_v, [pos], v * 16 + lane, mask=m)
        return cnt + plsc.all_reduce_population_count(m)[0]

    cnt = lax.fori_loop(0, BATCH // 16, scan_body, jnp.int32(0))
    full = lane >= 0
    plsc.store_scatter(hitc_v, [cnt + lane],
                       jnp.full((16,), jnp.int32(0x7FFFFFF0)), mask=full)
    plsc.store_scatter(hiti_v, [cnt + lane], BATCH + lane, mask=full)
    n_hit_groups = (cnt + 15) >> 4

    def fire_chunk(t, b, semc):
        s_tc = jnp.minimum(lo_tc + CHUNK_TC * t, S_CLAMP)
        off = pl.multiple_of(s_tc * 128, 128)
        pltpu.async_copy(qt_hbm.at[:, pl.ds(off, CHUNK_COLS)],
                         blk_v.at[b], semc)

    def process_chunk(t, b, semc, pend):
        pltpu.make_async_copy(qt_hbm.at[:, pl.ds(0, CHUNK_COLS)],
                              blk_v.at[b], semc).wait()
        s_tc = jnp.minimum(lo_tc + CHUNK_TC * t, S_CLAMP)
        sub_lo = s_tc * 128

        def p2_body(g, cnt2):
            hc = hitc_v[pl.ds(g * 16, 16)]
            hid = hiti_v[pl.ds(g * 16, 16)]
            m = (hc >= sub_lo) & (hc < sub_lo + CHUNK_COLS)
            mi = m.astype(jnp.int32)
            pos = cnt2 + plsc.cumsum(mi) - mi
            plsc.store_scatter(chc_v, [pos], hc - sub_lo, mask=m)
            plsc.store_scatter(chi_v, [pos], hid, mask=m)
            return cnt2 + plsc.all_reduce_population_count(m)[0]

        cnt2 = lax.fori_loop(0, n_hit_groups, p2_body, jnp.int32(0))
        plsc.store_scatter(chc_v, [cnt2 + lane], jnp.zeros((16,), jnp.int32),
                           mask=full)
        plsc.store_scatter(chi_v, [cnt2 + lane], BATCH + lane, mask=full)

        def ex_body(g, pend_in):
            def drain1(_, acc):
                pltpu.make_async_copy(qg_hbm.at[pl.ds(0, DIM)],
                                      rowbuf_v.at[0], semw).wait()
                return acc
            lax.fori_loop(0, pend_in, drain1, jnp.int32(0))
            hcv = chc_v[pl.ds(g * 16, 16)]
            hiv = chi_v[pl.ds(g * 16, 16)]
            for e in range(16):
                ccs = jnp.full((16,), hcv[e], jnp.int32)
                for k in range(4):
                    vals = plsc.load_gather(blk_v.at[b],
                                            [lane + (k * 16), ccs])
                    rowbuf_v[e, pl.ds(k * 16, 16)] = vals
                pltpu.async_copy(rowbuf_v.at[e],
                                 qg_hbm.at[pl.ds(hiv[e] * DIM, DIM)], semw)
            return jnp.int32(16)

        return lax.fori_loop(0, (cnt2 + 15) >> 4, ex_body, pend)

    fire_chunk(jnp.int32(0), 0, semc0)
    fire_chunk(jnp.int32(1), 1, semc1)

    def pair_body(p, pend):
        t0 = 2 * p
        pend = process_chunk(t0, 0, semc0, pend)
        fire_chunk(t0 + 2, 0, semc0)
        pend = process_chunk(t0 + 1, 1, semc1, pend)
        fire_chunk(t0 + 3, 1, semc1)
        return pend

    pend = lax.fori_loop(0, N_PAIRS, pair_body, jnp.int32(0))
    pltpu.make_async_copy(qt_hbm.at[:, pl.ds(0, CHUNK_COLS)],
                          blk_v.at[0], semc0).wait()
    pltpu.make_async_copy(qt_hbm.at[:, pl.ds(0, CHUNK_COLS)],
                          blk_v.at[1], semc1).wait()

    def drain_tail(_, acc):
        pltpu.make_async_copy(qg_hbm.at[pl.ds(0, DIM)],
                              rowbuf_v.at[0], semw).wait()
        return acc

    lax.fori_loop(0, pend, drain_tail, jnp.int32(0))


def _mf_kernel(model_hbm, qg_hbm, p_tab, w_hbm, b0_hbm, b1_hbm,
               out0_hbm, out1_hbm,
               midx_v, p_rows, q_rows, w_v, b0_v, b1_v,
               out0_v, out1_v, sem):
    wid = lax.axis_index("s") * NUM_CORES + lax.axis_index("c")
    base = wid * B_PER_W

    pltpu.sync_copy(w_hbm, w_v)
    pltpu.sync_copy(b0_hbm, b0_v)
    pltpu.sync_copy(b1_hbm, b1_v)
    for j in range(N_CHUNKS):
        off = base + j * IDX_CHUNK
        pltpu.sync_copy(model_hbm.at[pl.ds(off, IDX_CHUNK)], midx_v.at[j])

    copies = [pltpu.async_copy(qg_hbm.at[pl.ds(base * DIM, B_PER_W * DIM)],
                               q_rows, sem)]
    for j in range(N_CHUNKS):
        dst = pl.ds(j * IDX_CHUNK, IDX_CHUNK)
        copies.append(pltpu.async_copy(p_tab.at[midx_v.at[j]],
                                       p_rows.at[dst], sem))
    for cp in copies:
        cp.wait()

    w0 = [w_v[0, pl.ds(k * 16, 16)] for k in range(4)]
    w1 = [w_v[1, pl.ds(k * 16, 16)] for k in range(4)]
    b0 = b0_v[pl.ds(0, 16)]
    b1 = b1_v[pl.ds(0, 16)]
    lane = lax.iota(jnp.int32, 16)
    last = lane == 15

    def group_body(g, carry):
        for e in range(GROUP):
            i = g * GROUP + e
            h = [p_rows[i, pl.ds(k * 16, 16)]
                 * q_rows[pl.ds(i * DIM + k * 16, 16)]
                 for k in range(4)]
            s0 = h[0] * w0[0] + h[1] * w0[1] + h[2] * w0[2] + h[3] * w0[3]
            s1 = h[0] * w1[0] + h[1] * w1[1] + h[2] * w1[2] + h[3] * w1[3]
            r0 = plsc.cumsum(s0) + b0
            r1 = plsc.cumsum(s1) + b1
            idx = jnp.full((16,), i, jnp.int32)
            plsc.store_scatter(out0_v, [idx], r0, mask=last)
            plsc.store_scatter(out1_v, [idx], r1, mask=last)
        return carry

    lax.fori_loop(0, N_GROUPS, group_body, 0)

    pltpu.sync_copy(out0_v, out0_hbm.at[pl.ds(base, B_PER_W)])
    pltpu.sync_copy(out1_v, out1_hbm.at[pl.ds(base, B_PER_W)])


@jax.jit
def kernel(model, prompt, P, Q, W, b):
    mesh = plsc.VectorSubcoreMesh(core_axis_name="c", subcore_axis_name="s")
    qt = Q.T

    q_extract = functools.partial(
        pl.kernel,
        out_type=[jax.ShapeDtypeStruct((QG_ROWS * DIM,), jnp.float32)],
        mesh=mesh,
        scratch_types=[
            pltpu.VMEM((BATCH,), jnp.int32),
            pltpu.VMEM((HIT_CAP,), jnp.int32),
            pltpu.VMEM((HIT_CAP,), jnp.int32),
            pltpu.VMEM((CHIT_CAP,), jnp.int32),
            pltpu.VMEM((CHIT_CAP,), jnp.int32),
            pltpu.VMEM((2, DIM, CHUNK_COLS), jnp.float32),
            pltpu.VMEM((16, DIM), jnp.float32),
            pltpu.SemaphoreType.DMA,
            pltpu.SemaphoreType.DMA,
            pltpu.SemaphoreType.DMA,
        ],
        compiler_params=pltpu.CompilerParams(needs_layout_passes=False,
                                             use_tc_tiling_on_sc=True),
    )(_q_extract_kernel)
    (qg,) = q_extract(qt, prompt)

    b0_splat = jnp.full((16,), b[0], jnp.float32)
    b1_splat = jnp.full((16,), b[1], jnp.float32)
    mf = functools.partial(
        pl.kernel,
        out_type=[jax.ShapeDtypeStruct((BATCH,), jnp.float32),
                  jax.ShapeDtypeStruct((BATCH,), jnp.float32)],
        mesh=mesh,
        scratch_types=[
            pltpu.VMEM((N_CHUNKS, IDX_CHUNK), jnp.int32),
            pltpu.VMEM((B_PER_W, DIM), jnp.float32),
            pltpu.VMEM((B_PER_W * DIM,), jnp.float32),
            pltpu.VMEM((2, DIM), jnp.float32),
            pltpu.VMEM((16,), jnp.float32),
            pltpu.VMEM((16,), jnp.float32),
            pltpu.VMEM((B_PER_W,), jnp.float32),
            pltpu.VMEM((B_PER_W,), jnp.float32),
            pltpu.SemaphoreType.DMA,
        ],
        compiler_params=pltpu.CompilerParams(needs_layout_passes=False,
                                             use_tc_tiling_on_sc=False),
    )(_mf_kernel)
    out0, out1 = mf(model, qg, P, W, b0_splat, b1_splat)
    return jnp.stack([out0, out1], axis=1)
